# initial kernel scaffold (unmeasured)
import jax
import jax.numpy as jnp
from jax import lax
from jax.experimental import pallas as pl
from jax.experimental.pallas import tpu as pltpu


def kernel(
    x,
):
    def body(*refs):
        pass

    out_shape = jax.ShapeDtypeStruct(..., jnp.float32)
    return pl.pallas_call(body, out_shape=out_shape)(...)


# baseline (device time: 17454 ns/iter reference)
import jax
import jax.numpy as jnp
from jax import lax
from jax.experimental import pallas as pl
from jax.experimental.pallas import tpu as pltpu


def kernel(x):
    m_per, n = x.shape
    m_out = 2 * m_per

    def body(x_ref, out_ref, send_sem, recv_sem):
        my_x = lax.axis_index("x")
        my_y = lax.axis_index("y")
        other_y = 1 - my_y

        barrier_sem = pltpu.get_barrier_semaphore()
        pl.semaphore_signal(
            barrier_sem, inc=1,
            device_id=(my_x, other_y), device_id_type=pl.DeviceIdType.MESH,
        )
        pl.semaphore_wait(barrier_sem, 1)

        out_ref[pl.ds(my_y * m_per, m_per), :] = x_ref[:, :]

        rdma = pltpu.make_async_remote_copy(
            src_ref=x_ref,
            dst_ref=out_ref.at[pl.ds(my_y * m_per, m_per), :],
            send_sem=send_sem,
            recv_sem=recv_sem,
            device_id=(my_x, other_y),
            device_id_type=pl.DeviceIdType.MESH,
        )
        rdma.start()
        rdma.wait()

    return pl.pallas_call(
        body,
        out_shape=jax.ShapeDtypeStruct((m_out, n), x.dtype),
        in_specs=[pl.BlockSpec(memory_space=pltpu.VMEM)],
        out_specs=pl.BlockSpec(memory_space=pltpu.VMEM),
        scratch_shapes=[
            pltpu.SemaphoreType.DMA,
            pltpu.SemaphoreType.DMA,
        ],
        compiler_params=pltpu.CompilerParams(collective_id=0),
    )(x)


# device time: 15289 ns/iter; 1.1416x vs baseline; 1.1416x over previous
import jax
import jax.numpy as jnp
from jax import lax
from jax.experimental import pallas as pl
from jax.experimental.pallas import tpu as pltpu

K = 8


def kernel(x):
    m_per, n = x.shape
    m_out = 2 * m_per
    half = m_per // 2
    chunk = half // K

    def body(x_ref, out_ref, copy_sem, v_send, v_recv, h_send, h_recv):
        my_x = lax.axis_index("x")
        my_y = lax.axis_index("y")
        ox = 1 - my_x
        oy = 1 - my_y

        barrier_sem = pltpu.get_barrier_semaphore()
        for nbr in ((my_x, oy), (ox, my_y)):
            pl.semaphore_signal(
                barrier_sem, inc=1,
                device_id=nbr, device_id_type=pl.DeviceIdType.MESH,
            )
        pl.semaphore_wait(barrier_sem, 2)

        local = pltpu.make_async_copy(
            x_ref, out_ref.at[pl.ds(my_y * m_per, m_per), :], copy_sem,
        )
        local.start()

        v_rdmas = []
        for k in range(K):
            r = pltpu.make_async_remote_copy(
                src_ref=x_ref.at[pl.ds(my_x * half + k * chunk, chunk), :],
                dst_ref=out_ref.at[
                    pl.ds(my_y * m_per + my_x * half + k * chunk, chunk), :
                ],
                send_sem=v_send.at[k],
                recv_sem=v_recv.at[k],
                device_id=(my_x, oy),
                device_id_type=pl.DeviceIdType.MESH,
            )
            r.start()
            v_rdmas.append(r)

        h_rdmas = []
        for k in range(K):
            v_rdmas[k].wait_recv()
            landed = pl.ds(oy * m_per + my_x * half + k * chunk, chunk)
            f = pltpu.make_async_remote_copy(
                src_ref=out_ref.at[landed, :],
                dst_ref=out_ref.at[landed, :],
                send_sem=h_send.at[k],
                recv_sem=h_recv.at[k],
                device_id=(ox, my_y),
                device_id_type=pl.DeviceIdType.MESH,
            )
            f.start()
            h_rdmas.append(f)

        for k in range(K):
            h_rdmas[k].wait_recv()
        for k in range(K):
            v_rdmas[k].wait_send()
            h_rdmas[k].wait_send()
        local.wait()

    return pl.pallas_call(
        body,
        out_shape=jax.ShapeDtypeStruct((m_out, n), x.dtype),
        in_specs=[pl.BlockSpec(memory_space=pltpu.VMEM)],
        out_specs=pl.BlockSpec(memory_space=pltpu.VMEM),
        scratch_shapes=[
            pltpu.SemaphoreType.DMA,
            pltpu.SemaphoreType.DMA((K,)),
            pltpu.SemaphoreType.DMA((K,)),
            pltpu.SemaphoreType.DMA((K,)),
            pltpu.SemaphoreType.DMA((K,)),
        ],
        compiler_params=pltpu.CompilerParams(collective_id=0),
    )(x)


# device time: 15230 ns/iter; 1.1460x vs baseline; 1.0039x over previous
import jax
import jax.numpy as jnp
from jax import lax
from jax.experimental import pallas as pl
from jax.experimental.pallas import tpu as pltpu

K = 16


def kernel(x):
    m_per, n = x.shape
    m_out = 2 * m_per
    half = m_per // 2
    chunk = half // K

    def body(x_ref, out_ref, copy_sem, v_send, v_recv, h_send, h_recv, h_barrier):
        my_x = lax.axis_index("x")
        my_y = lax.axis_index("y")
        ox = 1 - my_x
        oy = 1 - my_y

        barrier_sem = pltpu.get_barrier_semaphore()
        pl.semaphore_signal(
            barrier_sem, inc=1,
            device_id=(my_x, oy), device_id_type=pl.DeviceIdType.MESH,
        )
        pl.semaphore_signal(
            h_barrier, inc=1,
            device_id=(ox, my_y), device_id_type=pl.DeviceIdType.MESH,
        )
        pl.semaphore_wait(barrier_sem, 1)

        local = pltpu.make_async_copy(
            x_ref, out_ref.at[pl.ds(my_y * m_per, m_per), :], copy_sem,
        )
        local.start()

        v_rdmas = []
        for k in range(K):
            r = pltpu.make_async_remote_copy(
                src_ref=x_ref.at[pl.ds(my_x * half + k * chunk, chunk), :],
                dst_ref=out_ref.at[
                    pl.ds(my_y * m_per + my_x * half + k * chunk, chunk), :
                ],
                send_sem=v_send.at[k],
                recv_sem=v_recv.at[k],
                device_id=(my_x, oy),
                device_id_type=pl.DeviceIdType.MESH,
            )
            r.start()
            v_rdmas.append(r)

        h_rdmas = []
        for k in range(K):
            v_rdmas[k].wait_recv()
            if k == 0:
                pl.semaphore_wait(h_barrier, 1)
            landed = pl.ds(oy * m_per + my_x * half + k * chunk, chunk)
            f = pltpu.make_async_remote_copy(
                src_ref=out_ref.at[landed, :],
                dst_ref=out_ref.at[landed, :],
                send_sem=h_send.at[k],
                recv_sem=h_recv.at[k],
                device_id=(ox, my_y),
                device_id_type=pl.DeviceIdType.MESH,
            )
            f.start()
            h_rdmas.append(f)

        for k in range(K):
            h_rdmas[k].wait_recv()
        for k in range(K):
            v_rdmas[k].wait_send()
            h_rdmas[k].wait_send()
        local.wait()

    return pl.pallas_call(
        body,
        out_shape=jax.ShapeDtypeStruct((m_out, n), x.dtype),
        in_specs=[pl.BlockSpec(memory_space=pltpu.VMEM)],
        out_specs=pl.BlockSpec(memory_space=pltpu.VMEM),
        scratch_shapes=[
            pltpu.SemaphoreType.DMA,
            pltpu.SemaphoreType.DMA((K,)),
            pltpu.SemaphoreType.DMA((K,)),
            pltpu.SemaphoreType.DMA((K,)),
            pltpu.SemaphoreType.DMA((K,)),
            pltpu.SemaphoreType.REGULAR,
        ],
        compiler_params=pltpu.CompilerParams(collective_id=0),
    )(x)
